# baseline (speedup 1.0000x reference)
import functools

import jax
import jax.numpy as jnp
from jax import lax
from jax.experimental import pallas as pl
from jax.experimental.pallas import tpu as pltpu
from jax.experimental.pallas import tpu_sc as plsc

D = 128
NC, NS = 2, 16
NW = NC * NS
CHUNK = 128
NBUF = 4

_F32 = jnp.float32


@functools.partial(jax.jit, static_argnames=("nc0", "nc1"))
def _sc_gather(h, idx2d, *, nc0, nc1):
    nc_max = max(nc0, nc1)
    b_rows = NS * (nc0 + nc1) * CHUNK
    dt = h.dtype
    w = h.shape[1]
    mesh = plsc.VectorSubcoreMesh(core_axis_name="c", subcore_axis_name="s")

    def body(h_hbm, idx_hbm, out_hbm, idx_v, bufs, gsems, wsems):
        cid = lax.axis_index("c")
        sid = lax.axis_index("s")
        my_nc = jnp.where(cid == 0, nc0, nc1)
        base_chunk = pl.multiple_of(
            jnp.where(cid == 0, sid * nc0, NS * nc0 + sid * nc1), 8)
        pltpu.sync_copy(idx_hbm.at[pl.ds(base_chunk, nc_max)], idx_v)

        def out_rows(j):
            return out_hbm.at[pl.ds(
                pl.multiple_of((base_chunk + j) * CHUNK, CHUNK), CHUNK)]

        for b in range(NBUF):
            pltpu.async_copy(h_hbm.at[idx_v.at[b]], bufs.at[b], gsems.at[b])

        def round4(i, carry):
            for b in range(NBUF):
                j = NBUF * i + b
                pltpu.make_async_copy(
                    h_hbm.at[idx_v.at[j]], bufs.at[b], gsems.at[b]).wait()
                pltpu.async_copy(bufs.at[b], out_rows(j), wsems.at[b])
            for b in range(NBUF):
                jn = NBUF * (i + 1) + b

                @pl.when(jn < my_nc)
                def _():
                    pltpu.make_async_copy(
                        bufs.at[b], out_rows(jn - NBUF), wsems.at[b]).wait()
                    pltpu.async_copy(
                        h_hbm.at[idx_v.at[jn]], bufs.at[b], gsems.at[b])

            return carry

        lax.fori_loop(0, my_nc // NBUF, round4, 0)
        for b in range(NBUF):
            pltpu.make_async_copy(
                bufs.at[b], out_rows(0), wsems.at[b]).wait()

    fn = pl.kernel(
        body,
        out_type=jax.ShapeDtypeStruct((b_rows, w), dt),
        mesh=mesh,
        scratch_types=[
            pltpu.VMEM((nc_max, CHUNK), jnp.int32),
            pltpu.VMEM((NBUF, CHUNK, w), dt),
            pltpu.SemaphoreType.DMA((NBUF,)),
            pltpu.SemaphoreType.DMA((NBUF,)),
        ],
    )
    return fn(h, idx2d)


def _dot(a, b):
    return jnp.dot(a, b, preferred_element_type=_F32)


def _tail(a, w2, b2, w3, b3):
    a = jnp.maximum(a, 0.0)
    a = jnp.maximum(_dot(a, w2) + b2, 0.0)
    a = jnp.maximum(_dot(a, w3) + b3, 0.0)
    return a


def _full_spec(arr):
    return pl.BlockSpec(arr.shape, lambda g: (0,) * arr.ndim)


def _level_mlp(G, col_offsets, seq, heads, n_rows, block_rows):
    k = len(col_offsets)
    (w1, b1), (w2, b2), (w3, b3) = seq
    wh = jnp.concatenate([w for (w, _) in heads], axis=1)
    bh = jnp.concatenate([b for (_, b) in heads])[None, :]
    b1, b2, b3 = b1[None, :], b2[None, :], b3[None, :]
    nh = wh.shape[1]

    def body(*refs):
        col_refs, (w1r, b1r, w2r, b2r, w3r, b3r, whr, bhr, out) = refs[:k], refs[k:]
        w1v = w1r[...]
        accf = None
        accr = None
        for i in range(k):
            cv = col_refs[i][...]
            df = _dot(cv, w1v[i * D:(i + 1) * D])
            dr = _dot(cv, w1v[(k - 1 - i) * D:(k - i) * D])
            accf = df if accf is None else accf + df
            accr = dr if accr is None else accr + dr
        zf = _tail(accf + b1r[...], w2r[...], b2r[...], w3r[...], b3r[...])
        zr = _tail(accr + b1r[...], w2r[...], b2r[...], w3r[...], b3r[...])
        z = zf + zr
        out[...] = _dot(z, whr[...]) + bhr[...]

    assert n_rows % block_rows == 0
    col_specs = []
    for off in col_offsets:
        assert off % block_rows == 0
        col_specs.append(pl.BlockSpec(
            (block_rows, D),
            functools.partial(lambda o, g: (o + g, 0), off // block_rows)))
    weights = (w1, b1, w2, b2, w3, b3, wh, bh)
    return pl.pallas_call(
        body,
        grid=(n_rows // block_rows,),
        in_specs=col_specs + [_full_spec(w) for w in weights],
        out_specs=pl.BlockSpec((block_rows, nh), lambda g: (g, 0)),
        out_shape=jax.ShapeDtypeStruct((n_rows, nh), _F32),
    )(*([G] * k + list(weights)))


def _level1_mlp(h, seq, heads, block_rows):
    (w1, b1), (w2, b2), (w3, b3) = seq
    wh = jnp.concatenate([w for (w, _) in heads], axis=1)
    bh = jnp.concatenate([b for (_, b) in heads])[None, :]
    b1, b2, b3 = b1[None, :], b2[None, :], b3[None, :]
    nh = wh.shape[1]
    n_rows = h.shape[0]

    def body(hr, w1r, b1r, w2r, b2r, w3r, b3r, whr, bhr, out):
        a = _tail(_dot(hr[...], w1r[...]) + b1r[...],
                  w2r[...], b2r[...], w3r[...], b3r[...])
        out[...] = _dot(a, whr[...]) + bhr[...]

    assert n_rows % block_rows == 0
    weights = (w1, b1, w2, b2, w3, b3, wh, bh)
    return pl.pallas_call(
        body,
        grid=(n_rows // block_rows,),
        in_specs=[pl.BlockSpec((block_rows, D), lambda g: (g, 0))]
                 + [_full_spec(w) for w in weights],
        out_specs=pl.BlockSpec((block_rows, nh), lambda g: (g, 0)),
        out_shape=jax.ShapeDtypeStruct((n_rows, nh), _F32),
    )(h, *weights)


def _pad_idx(flat):
    b = flat.shape[0]
    tot = -(-b // (NS * CHUNK * 16)) * 16
    nc0 = tot // 2
    nc1 = tot - nc0
    assert nc1 % 8 == 0 and nc0 % 8 == 0 and min(nc0, nc1) >= NBUF
    idx_len = (NS * nc0 + (NS - 1) * nc1 + max(nc0, nc1)) * CHUNK
    flat = jnp.concatenate([flat, jnp.zeros((idx_len - b,), jnp.int32)])
    return flat.reshape(-1, CHUNK), nc0, nc1


def kernel(h, idx2, idx3, idx4, params):
    n2, n3, n4 = idx2.shape[0], idx3.shape[0], idx4.shape[0]

    flat = jnp.concatenate(
        [idx2.T.reshape(-1), idx3.T.reshape(-1), idx4.T.reshape(-1)])
    idx2d, nc0, nc1 = _pad_idx(flat)

    G = _sc_gather(h, idx2d, nc0=nc0, nc1=nc1)

    off2 = (0, n2)
    off3 = (2 * n2, 2 * n2 + n3, 2 * n2 + 2 * n3)
    base4 = 2 * n2 + 3 * n3
    off4 = (base4, base4 + n4, base4 + 2 * n4, base4 + 3 * n4)

    p = params
    o1 = _level1_mlp(h, p["seq1"],
                     (p["head_1_sigma"], p["head_1_epsilon"], p["head_1_q"]),
                     block_rows=1000)
    o2 = _level_mlp(G, off2, p["seq2"], (p["head_2_k"], p["head_2_eq"]),
                    n_rows=n2, block_rows=1000)
    o3 = _level_mlp(G, off3, p["seq3"], (p["head_3_k"], p["head_3_eq"]),
                    n_rows=n3, block_rows=1000)
    o4 = _level_mlp(G, off4, p["seq4"], (p["head_4_k"], p["head_4_eq"]),
                    n_rows=n4, block_rows=1000)

    def col(o, i):
        return jax.lax.optimization_barrier(o[:, i]).reshape(-1, 1)

    return (col(o1, 0), col(o1, 1), col(o1, 2),
            col(o2, 0), col(o2, 1),
            col(o3, 0), col(o3, 1),
            col(o4, 0), col(o4, 1))


# speedup vs baseline: 1.1760x; 1.1760x over previous
import functools

import jax
import jax.numpy as jnp
from jax import lax
from jax.experimental import pallas as pl
from jax.experimental.pallas import tpu as pltpu
from jax.experimental.pallas import tpu_sc as plsc

D = 128
NC, NS = 2, 16
NW = NC * NS
CHUNK = 128
NBUF = 4

_F32 = jnp.float32


@functools.partial(jax.jit, static_argnames=("nc0", "nc1"))
def _sc_gather(h, idx2d, *, nc0, nc1):
    nc_max = max(nc0, nc1)
    b_rows = NS * (nc0 + nc1) * CHUNK
    dt = h.dtype
    w = h.shape[1]
    mesh = plsc.VectorSubcoreMesh(core_axis_name="c", subcore_axis_name="s")

    def body(h_hbm, idx_hbm, out_hbm, idx_v, bufs, gsems, wsems):
        cid = lax.axis_index("c")
        sid = lax.axis_index("s")
        my_nc = jnp.where(cid == 0, nc0, nc1)
        base_chunk = pl.multiple_of(
            jnp.where(cid == 0, sid * nc0, NS * nc0 + sid * nc1), 8)
        pltpu.sync_copy(idx_hbm.at[pl.ds(base_chunk, nc_max)], idx_v)

        def out_rows(j):
            return out_hbm.at[pl.ds(
                pl.multiple_of((base_chunk + j) * CHUNK, CHUNK), CHUNK)]

        for b in range(NBUF):
            pltpu.async_copy(h_hbm.at[idx_v.at[b]], bufs.at[b], gsems.at[b])

        def round4(i, carry):
            for b in range(NBUF):
                j = NBUF * i + b
                pltpu.make_async_copy(
                    h_hbm.at[idx_v.at[j]], bufs.at[b], gsems.at[b]).wait()
                pltpu.async_copy(bufs.at[b], out_rows(j), wsems.at[b])
            for b in range(NBUF):
                jn = NBUF * (i + 1) + b

                @pl.when(jn < my_nc)
                def _():
                    pltpu.make_async_copy(
                        bufs.at[b], out_rows(jn - NBUF), wsems.at[b]).wait()
                    pltpu.async_copy(
                        h_hbm.at[idx_v.at[jn]], bufs.at[b], gsems.at[b])

            return carry

        lax.fori_loop(0, my_nc // NBUF, round4, 0)
        for b in range(NBUF):
            pltpu.make_async_copy(
                bufs.at[b], out_rows(0), wsems.at[b]).wait()

    fn = pl.kernel(
        body,
        out_type=jax.ShapeDtypeStruct((b_rows, w), dt),
        mesh=mesh,
        scratch_types=[
            pltpu.VMEM((nc_max, CHUNK), jnp.int32),
            pltpu.VMEM((NBUF, CHUNK, w), dt),
            pltpu.SemaphoreType.DMA((NBUF,)),
            pltpu.SemaphoreType.DMA((NBUF,)),
        ],
    )
    return fn(h, idx2d)


def _dot(a, b):
    return jnp.dot(a, b, preferred_element_type=_F32)


def _tail(a, w2, b2, w3, b3):
    a = jnp.maximum(a, 0.0)
    a = jnp.maximum(_dot(a, w2) + b2, 0.0)
    a = jnp.maximum(_dot(a, w3) + b3, 0.0)
    return a


def _full_spec(arr):
    return pl.BlockSpec(arr.shape, lambda g: (0,) * arr.ndim)


def _level_mlp(G, col_offsets, seq, heads, n_rows, block_rows):
    k = len(col_offsets)
    (w1, b1), (w2, b2), (w3, b3) = seq
    wh = jnp.concatenate([w for (w, _) in heads], axis=1)
    bh = jnp.concatenate([b for (_, b) in heads])[None, :]
    b1, b2, b3 = b1[None, :], b2[None, :], b3[None, :]
    nh = wh.shape[1]

    def body(*refs):
        col_refs, (w1r, b1r, w2r, b2r, w3r, b3r, whr, bhr, out) = refs[:k], refs[k:]
        w1v = w1r[...]
        accf = None
        accr = None
        for i in range(k):
            cv = col_refs[i][...]
            df = _dot(cv, w1v[i * D:(i + 1) * D])
            dr = _dot(cv, w1v[(k - 1 - i) * D:(k - i) * D])
            accf = df if accf is None else accf + df
            accr = dr if accr is None else accr + dr
        zf = _tail(accf + b1r[...], w2r[...], b2r[...], w3r[...], b3r[...])
        zr = _tail(accr + b1r[...], w2r[...], b2r[...], w3r[...], b3r[...])
        z = zf + zr
        out[...] = _dot(z, whr[...]) + bhr[...]

    assert n_rows % block_rows == 0
    col_specs = []
    for off in col_offsets:
        assert off % block_rows == 0
        col_specs.append(pl.BlockSpec(
            (block_rows, D),
            functools.partial(lambda o, g: (o + g, 0), off // block_rows)))
    weights = (w1, b1, w2, b2, w3, b3, wh, bh)
    return pl.pallas_call(
        body,
        grid=(n_rows // block_rows,),
        in_specs=col_specs + [_full_spec(w) for w in weights],
        out_specs=pl.BlockSpec((block_rows, nh), lambda g: (g, 0)),
        out_shape=jax.ShapeDtypeStruct((n_rows, nh), _F32),
    )(*([G] * k + list(weights)))


def _level1_mlp(h, seq, heads, block_rows):
    (w1, b1), (w2, b2), (w3, b3) = seq
    wh = jnp.concatenate([w for (w, _) in heads], axis=1)
    bh = jnp.concatenate([b for (_, b) in heads])[None, :]
    b1, b2, b3 = b1[None, :], b2[None, :], b3[None, :]
    nh = wh.shape[1]
    n_rows = h.shape[0]

    def body(hr, w1r, b1r, w2r, b2r, w3r, b3r, whr, bhr, out):
        a = _tail(_dot(hr[...], w1r[...]) + b1r[...],
                  w2r[...], b2r[...], w3r[...], b3r[...])
        out[...] = _dot(a, whr[...]) + bhr[...]

    assert n_rows % block_rows == 0
    weights = (w1, b1, w2, b2, w3, b3, wh, bh)
    return pl.pallas_call(
        body,
        grid=(n_rows // block_rows,),
        in_specs=[pl.BlockSpec((block_rows, D), lambda g: (g, 0))]
                 + [_full_spec(w) for w in weights],
        out_specs=pl.BlockSpec((block_rows, nh), lambda g: (g, 0)),
        out_shape=jax.ShapeDtypeStruct((n_rows, nh), _F32),
    )(h, *weights)


def _pad_idx(flat):
    b = flat.shape[0]
    tot = -(-b // (NS * CHUNK * 16)) * 16
    nc0 = (int(round(tot * 0.55)) // 8) * 8
    nc1 = tot - nc0
    assert nc1 % 8 == 0 and nc0 % 8 == 0 and min(nc0, nc1) >= NBUF
    idx_len = (NS * nc0 + (NS - 1) * nc1 + max(nc0, nc1)) * CHUNK
    flat = jnp.concatenate([flat, jnp.zeros((idx_len - b,), jnp.int32)])
    return flat.reshape(-1, CHUNK), nc0, nc1


def kernel(h, idx2, idx3, idx4, params):
    n2, n3, n4 = idx2.shape[0], idx3.shape[0], idx4.shape[0]

    flat = jnp.concatenate(
        [idx2.T.reshape(-1), idx3.T.reshape(-1), idx4.T.reshape(-1)])
    idx2d, nc0, nc1 = _pad_idx(flat)

    G = _sc_gather(h, idx2d, nc0=nc0, nc1=nc1)

    off2 = (0, n2)
    off3 = (2 * n2, 2 * n2 + n3, 2 * n2 + 2 * n3)
    base4 = 2 * n2 + 3 * n3
    off4 = (base4, base4 + n4, base4 + 2 * n4, base4 + 3 * n4)

    p = params
    o1 = _level1_mlp(h, p["seq1"],
                     (p["head_1_sigma"], p["head_1_epsilon"], p["head_1_q"]),
                     block_rows=2000)
    o2 = _level_mlp(G, off2, p["seq2"], (p["head_2_k"], p["head_2_eq"]),
                    n_rows=n2, block_rows=2000)
    o3 = _level_mlp(G, off3, p["seq3"], (p["head_3_k"], p["head_3_eq"]),
                    n_rows=n3, block_rows=2000)
    o4 = _level_mlp(G, off4, p["seq4"], (p["head_4_k"], p["head_4_eq"]),
                    n_rows=n4, block_rows=2000)

    def col(o, i):
        return jax.lax.optimization_barrier(o[:, i]).reshape(-1, 1)

    return (col(o1, 0), col(o1, 1), col(o1, 2),
            col(o2, 0), col(o2, 1),
            col(o3, 0), col(o3, 1),
            col(o4, 0), col(o4, 1))


# speedup vs baseline: 1.1829x; 1.0058x over previous
import functools

import jax
import jax.numpy as jnp
from jax import lax
from jax.experimental import pallas as pl
from jax.experimental.pallas import tpu as pltpu
from jax.experimental.pallas import tpu_sc as plsc

D = 128
NC, NS = 2, 16
NW = NC * NS
CHUNK = 128
NBUF = 4

_F32 = jnp.float32


@functools.partial(jax.jit, static_argnames=("nc0", "nc1"))
def _sc_gather(h, idx2d, *, nc0, nc1):
    nc_max = max(nc0, nc1)
    b_rows = NS * (nc0 + nc1) * CHUNK
    dt = h.dtype
    w = h.shape[1]
    mesh = plsc.VectorSubcoreMesh(core_axis_name="c", subcore_axis_name="s")

    def body(h_hbm, idx_hbm, out_hbm, idx_v, bufs, gsems, wsems):
        cid = lax.axis_index("c")
        sid = lax.axis_index("s")
        my_nc = jnp.where(cid == 0, nc0, nc1)
        base_chunk = pl.multiple_of(
            jnp.where(cid == 0, sid * nc0, NS * nc0 + sid * nc1), 8)
        pltpu.sync_copy(idx_hbm.at[pl.ds(base_chunk, nc_max)], idx_v)

        def out_rows(j):
            return out_hbm.at[pl.ds(
                pl.multiple_of((base_chunk + j) * CHUNK, CHUNK), CHUNK)]

        for b in range(NBUF):
            pltpu.async_copy(h_hbm.at[idx_v.at[b]], bufs.at[b], gsems.at[b])

        def round4(i, carry):
            for b in range(NBUF):
                j = NBUF * i + b
                pltpu.make_async_copy(
                    h_hbm.at[idx_v.at[j]], bufs.at[b], gsems.at[b]).wait()
                pltpu.async_copy(bufs.at[b], out_rows(j), wsems.at[b])
            for b in range(NBUF):
                jn = NBUF * (i + 1) + b

                @pl.when(jn < my_nc)
                def _():
                    pltpu.make_async_copy(
                        bufs.at[b], out_rows(jn - NBUF), wsems.at[b]).wait()
                    pltpu.async_copy(
                        h_hbm.at[idx_v.at[jn]], bufs.at[b], gsems.at[b])

            return carry

        lax.fori_loop(0, my_nc // NBUF, round4, 0)
        for b in range(NBUF):
            pltpu.make_async_copy(
                bufs.at[b], out_rows(0), wsems.at[b]).wait()

    fn = pl.kernel(
        body,
        out_type=jax.ShapeDtypeStruct((b_rows, w), dt),
        mesh=mesh,
        scratch_types=[
            pltpu.VMEM((nc_max, CHUNK), jnp.int32),
            pltpu.VMEM((NBUF, CHUNK, w), dt),
            pltpu.SemaphoreType.DMA((NBUF,)),
            pltpu.SemaphoreType.DMA((NBUF,)),
        ],
    )
    return fn(h, idx2d)


def _dot(a, b):
    return jnp.dot(a, b, preferred_element_type=_F32)


def _tail(a, w2, b2, w3, b3):
    a = jnp.maximum(a, 0.0)
    a = jnp.maximum(_dot(a, w2) + b2, 0.0)
    a = jnp.maximum(_dot(a, w3) + b3, 0.0)
    return a


def _full_spec(arr):
    return pl.BlockSpec(arr.shape, lambda g: (0,) * arr.ndim)


def _level_mlp(G, col_offsets, seq, heads, n_rows, block_rows):
    k = len(col_offsets)
    (w1, b1), (w2, b2), (w3, b3) = seq
    wh = jnp.concatenate([w for (w, _) in heads], axis=1)
    bh = jnp.concatenate([b for (_, b) in heads])[None, :]
    b1, b2, b3 = b1[None, :], b2[None, :], b3[None, :]
    nh = wh.shape[1]

    def body(*refs):
        col_refs, (w1r, b1r, w2r, b2r, w3r, b3r, whr, bhr, out) = refs[:k], refs[k:]
        w1v = w1r[...]
        accf = None
        accr = None
        for i in range(k):
            cv = col_refs[i][...]
            df = _dot(cv, w1v[i * D:(i + 1) * D])
            dr = _dot(cv, w1v[(k - 1 - i) * D:(k - i) * D])
            accf = df if accf is None else accf + df
            accr = dr if accr is None else accr + dr
        zf = _tail(accf + b1r[...], w2r[...], b2r[...], w3r[...], b3r[...])
        zr = _tail(accr + b1r[...], w2r[...], b2r[...], w3r[...], b3r[...])
        z = zf + zr
        out[...] = _dot(z, whr[...]) + bhr[...]

    assert n_rows % block_rows == 0
    col_specs = []
    for off in col_offsets:
        assert off % block_rows == 0
        col_specs.append(pl.BlockSpec(
            (block_rows, D),
            functools.partial(lambda o, g: (o + g, 0), off // block_rows)))
    weights = (w1, b1, w2, b2, w3, b3, wh, bh)
    return pl.pallas_call(
        body,
        grid=(n_rows // block_rows,),
        in_specs=col_specs + [_full_spec(w) for w in weights],
        out_specs=pl.BlockSpec((block_rows, nh), lambda g: (g, 0)),
        out_shape=jax.ShapeDtypeStruct((n_rows, nh), _F32),
    )(*([G] * k + list(weights)))


def _level1_mlp(h, seq, heads, block_rows):
    (w1, b1), (w2, b2), (w3, b3) = seq
    wh = jnp.concatenate([w for (w, _) in heads], axis=1)
    bh = jnp.concatenate([b for (_, b) in heads])[None, :]
    b1, b2, b3 = b1[None, :], b2[None, :], b3[None, :]
    nh = wh.shape[1]
    n_rows = h.shape[0]

    def body(hr, w1r, b1r, w2r, b2r, w3r, b3r, whr, bhr, out):
        a = _tail(_dot(hr[...], w1r[...]) + b1r[...],
                  w2r[...], b2r[...], w3r[...], b3r[...])
        out[...] = _dot(a, whr[...]) + bhr[...]

    assert n_rows % block_rows == 0
    weights = (w1, b1, w2, b2, w3, b3, wh, bh)
    return pl.pallas_call(
        body,
        grid=(n_rows // block_rows,),
        in_specs=[pl.BlockSpec((block_rows, D), lambda g: (g, 0))]
                 + [_full_spec(w) for w in weights],
        out_specs=pl.BlockSpec((block_rows, nh), lambda g: (g, 0)),
        out_shape=jax.ShapeDtypeStruct((n_rows, nh), _F32),
    )(h, *weights)


def _pad_idx(flat):
    b = flat.shape[0]
    tot = -(-b // (NS * CHUNK * 16)) * 16
    nc0 = tot // 2
    nc1 = tot - nc0
    assert nc1 % 8 == 0 and nc0 % 8 == 0 and min(nc0, nc1) >= NBUF
    idx_len = (NS * nc0 + (NS - 1) * nc1 + max(nc0, nc1)) * CHUNK
    flat = jnp.concatenate([flat, jnp.zeros((idx_len - b,), jnp.int32)])
    return flat.reshape(-1, CHUNK), nc0, nc1


def kernel(h, idx2, idx3, idx4, params):
    n2, n3, n4 = idx2.shape[0], idx3.shape[0], idx4.shape[0]

    flat = jnp.concatenate(
        [idx2.T.reshape(-1), idx3.T.reshape(-1), idx4.T.reshape(-1)])
    idx2d, nc0, nc1 = _pad_idx(flat)

    G = _sc_gather(h, idx2d, nc0=nc0, nc1=nc1)

    off2 = (0, n2)
    off3 = (2 * n2, 2 * n2 + n3, 2 * n2 + 2 * n3)
    base4 = 2 * n2 + 3 * n3
    off4 = (base4, base4 + n4, base4 + 2 * n4, base4 + 3 * n4)

    p = params
    o1 = _level1_mlp(h, p["seq1"],
                     (p["head_1_sigma"], p["head_1_epsilon"], p["head_1_q"]),
                     block_rows=2000)
    o2 = _level_mlp(G, off2, p["seq2"], (p["head_2_k"], p["head_2_eq"]),
                    n_rows=n2, block_rows=2000)
    o3 = _level_mlp(G, off3, p["seq3"], (p["head_3_k"], p["head_3_eq"]),
                    n_rows=n3, block_rows=2000)
    o4 = _level_mlp(G, off4, p["seq4"], (p["head_4_k"], p["head_4_eq"]),
                    n_rows=n4, block_rows=2000)

    def col(o, i):
        return jax.lax.optimization_barrier(o[:, i]).reshape(-1, 1)

    return (col(o1, 0), col(o1, 1), col(o1, 2),
            col(o2, 0), col(o2, 1),
            col(o3, 0), col(o3, 1),
            col(o4, 0), col(o4, 1))
